# baseline (speedup 1.0000x reference)
import functools
import math

import jax
import jax.numpy as jnp
from jax import lax
from jax.experimental import pallas as pl
from jax.experimental.pallas import tpu as pltpu
from jax.experimental.pallas import tpu_sc as plsc

D_MODEL = 64
SCALE = math.sqrt(D_MODEL)
NC, NS, L = 2, 16, 16
NW = NC * NS

B_TOTAL = 4096 * 200
BPW = B_TOTAL // NW
CHUNK = 128
NCHUNK = BPW // CHUNK
NBUF = 4

_mesh = plsc.VectorSubcoreMesh(
    core_axis_name="c", subcore_axis_name="s", num_cores=NC, num_subcores=NS
)


@functools.partial(
    pl.kernel,
    out_type=jax.ShapeDtypeStruct((B_TOTAL, 2 * D_MODEL), jnp.float32),
    mesh=_mesh,
    scratch_types=[
        pltpu.VMEM((BPW,), jnp.int32),
        pltpu.VMEM((NBUF, CHUNK, D_MODEL // 2), jnp.float32),
        pltpu.VMEM((NBUF, CHUNK, D_MODEL // 2), jnp.float32),
        pltpu.VMEM((NBUF, CHUNK, 2 * D_MODEL), jnp.float32),
        pltpu.SemaphoreType.DMA((NBUF,)),
        pltpu.SemaphoreType.DMA((NBUF,)),
        pltpu.SemaphoreType.DMA((NBUF,)),
    ],
    compiler_params=pltpu.CompilerParams(use_tc_tiling_on_sc=False),
)
def _emb_lookup(x_hbm, lutl_hbm, luth_hbm, out_hbm, idx_v, rowsl, rowsh, wrows, gseml, gsemh, osem):
    wid = lax.axis_index("s") * NC + lax.axis_index("c")
    base = wid * BPW
    pltpu.sync_copy(x_hbm.at[pl.ds(base, BPW)], idx_v)

    def gather(g, b):
        pltpu.async_copy(
            lutl_hbm.at[idx_v.at[pl.ds(g * CHUNK, CHUNK)]],
            rowsl.at[b], gseml.at[b]
        )
        pltpu.async_copy(
            luth_hbm.at[idx_v.at[pl.ds(g * CHUNK, CHUNK)]],
            rowsh.at[b], gsemh.at[b]
        )

    def gather_wait(g, b):
        pltpu.make_async_copy(
            lutl_hbm.at[idx_v.at[pl.ds(g * CHUNK, CHUNK)]],
            rowsl.at[b], gseml.at[b]
        ).wait()
        pltpu.make_async_copy(
            luth_hbm.at[idx_v.at[pl.ds(g * CHUNK, CHUNK)]],
            rowsh.at[b], gsemh.at[b]
        ).wait()

    def outcopy(g, b):
        return pltpu.async_copy(
            wrows.at[b], out_hbm.at[pl.ds(base + g * CHUNK, CHUNK)], osem.at[b]
        )

    def outcopy_wait(g, b):
        pltpu.make_async_copy(
            wrows.at[b], out_hbm.at[pl.ds(base + g * CHUNK, CHUNK)], osem.at[b]
        ).wait()

    def scale(b):
        @plsc.parallel_loop(0, CHUNK, 1, unroll=4)
        def _row(r):
            for l in range(D_MODEL // (2 * L)):
                v = rowsl[b, r, pl.ds(l * L, L)]
                wrows[b, r, pl.ds(l * L, L)] = v * SCALE
            for l in range(D_MODEL // (2 * L)):
                v = rowsh[b, r, pl.ds(l * L, L)]
                wrows[b, r, pl.ds(D_MODEL // 2 + l * L, L)] = v * SCALE

    gather(0, 0)
    gather(1, 1)

    @pl.loop(0, NCHUNK // NBUF)
    def _quad(t):
        g0 = t * NBUF
        for b in range(NBUF):
            g = g0 + b
            hb = (b + 2) % NBUF
            if b < 2:
                @pl.when(t > 0)
                def _():
                    outcopy_wait(g - 2, hb)
                gather(g + 2, hb)
            else:
                outcopy_wait(g - 2, hb)
                @pl.when(g + 2 < NCHUNK)
                def _():
                    gather(g + 2, hb)
            gather_wait(g, b)
            scale(b)
            outcopy(g, b)

    outcopy_wait(NCHUNK - 2, (NCHUNK - 2) % NBUF)
    outcopy_wait(NCHUNK - 1, (NCHUNK - 1) % NBUF)


def kernel(x, lut):
    lutl = lut[:, : D_MODEL // 2]
    luth = lut[:, D_MODEL // 2:]
    out = _emb_lookup(x.reshape(-1).astype(jnp.int32), lutl, luth)
    return out[:, :D_MODEL].reshape(x.shape + (D_MODEL,))


# speedup vs baseline: 1.7728x; 1.7728x over previous
import functools
import math

import jax
import jax.numpy as jnp
from jax import lax
from jax.experimental import pallas as pl
from jax.experimental.pallas import tpu as pltpu
from jax.experimental.pallas import tpu_sc as plsc

D_MODEL = 64
SCALE = math.sqrt(D_MODEL)
NC, NS, L = 2, 16, 16
NW = NC * NS

B_TOTAL = 4096 * 200
BPW = B_TOTAL // NW
CHUNK = 128
NCHUNK = BPW // CHUNK
NBUF = 4

_mesh = plsc.VectorSubcoreMesh(
    core_axis_name="c", subcore_axis_name="s", num_cores=NC, num_subcores=NS
)


@functools.partial(
    pl.kernel,
    out_type=jax.ShapeDtypeStruct((B_TOTAL, 2 * D_MODEL), jnp.float32),
    mesh=_mesh,
    scratch_types=[
        pltpu.VMEM((BPW,), jnp.int32),
        pltpu.VMEM((NBUF, CHUNK, D_MODEL), jnp.float32),
        pltpu.VMEM((NBUF, CHUNK, 2 * D_MODEL), jnp.float32),
        pltpu.SemaphoreType.DMA((NBUF,)),
        pltpu.SemaphoreType.DMA((NBUF,)),
    ],
    compiler_params=pltpu.CompilerParams(use_tc_tiling_on_sc=False),
)
def _emb_lookup(x_hbm, lut_hbm, out_hbm, idx_v, rows, wrows, gsem, osem):
    wid = lax.axis_index("s") * NC + lax.axis_index("c")
    base = wid * BPW
    pltpu.sync_copy(x_hbm.at[pl.ds(base, BPW)], idx_v)

    def gather(g, b):
        return pltpu.async_copy(
            lut_hbm.at[idx_v.at[pl.ds(g * CHUNK, CHUNK)]],
            rows.at[b], gsem.at[b]
        )

    def gather_wait(g, b):
        pltpu.make_async_copy(
            lut_hbm.at[idx_v.at[pl.ds(g * CHUNK, CHUNK)]],
            rows.at[b], gsem.at[b]
        ).wait()

    def outcopy(g, b):
        return pltpu.async_copy(
            wrows.at[b], out_hbm.at[pl.ds(base + g * CHUNK, CHUNK)], osem.at[b]
        )

    def outcopy_wait(g, b):
        pltpu.make_async_copy(
            wrows.at[b], out_hbm.at[pl.ds(base + g * CHUNK, CHUNK)], osem.at[b]
        ).wait()

    def scale(b):
        @plsc.parallel_loop(0, CHUNK, 1, unroll=4)
        def _row(r):
            for l in range(D_MODEL // L):
                v = rows[b, r, pl.ds(l * L, L)]
                wrows[b, r, pl.ds(l * L, L)] = v * SCALE

    gather(0, 0)
    gather(1, 1)

    @pl.loop(0, NCHUNK // NBUF)
    def _quad(t):
        g0 = t * NBUF
        for b in range(NBUF):
            g = g0 + b
            hb = (b + 2) % NBUF
            if b < 2:
                @pl.when(t > 0)
                def _():
                    outcopy_wait(g - 2, hb)
                gather(g + 2, hb)
            else:
                outcopy_wait(g - 2, hb)
                @pl.when(g + 2 < NCHUNK)
                def _():
                    gather(g + 2, hb)
            gather_wait(g, b)
            scale(b)
            outcopy(g, b)

    outcopy_wait(NCHUNK - 2, (NCHUNK - 2) % NBUF)
    outcopy_wait(NCHUNK - 1, (NCHUNK - 1) % NBUF)


def kernel(x, lut):
    out = _emb_lookup(x.reshape(-1).astype(jnp.int32), lut)
    return out[:, :D_MODEL].reshape(x.shape + (D_MODEL,))
